# baseline (speedup 1.0000x reference)
import functools

import jax
import jax.numpy as jnp
from jax import lax
from jax.experimental import pallas as pl
from jax.experimental.pallas import tpu as pltpu
from jax.experimental.pallas import tpu_sc as plsc


def _build_dice_kernel(N, E, n_workers, lanes):
    C = 128
    per_w = N // n_workers
    n_chunks = per_w // C
    groups = C // lanes
    evecs = E // lanes

    mesh = plsc.VectorSubcoreMesh(core_axis_name="c", subcore_axis_name="s")
    nc = mesh.num_cores

    @functools.partial(
        pl.kernel,
        out_type=jax.ShapeDtypeStruct((N,), jnp.float32),
        mesh=mesh,
        compiler_params=pltpu.CompilerParams(use_tc_tiling_on_sc=False),
        scratch_types=[
            pltpu.VMEM((C,), jnp.int32),
            pltpu.VMEM((C,), jnp.int32),
            pltpu.VMEM((C, E), jnp.float32),
            pltpu.VMEM((C, E), jnp.float32),
            pltpu.VMEM((C, E), jnp.float32),
            pltpu.VMEM((C, E), jnp.float32),
            pltpu.VMEM((C,), jnp.float32),
            pltpu.SemaphoreType.DMA,
        ],
    )
    def dice(user_r, item_r, ui_r, up_r, ii_r, ip_r, out_r,
             idx_u, idx_i, uiv, upv, iiv, ipv, outv, sem):
        wid = lax.axis_index("s") * nc + lax.axis_index("c")
        base = wid * per_w
        lane = lax.iota(jnp.int32, lanes)
        xperms = [lane ^ k for k in (8, 4, 2, 1)]

        @pl.loop(0, n_chunks)
        def chunk_body(ci):
            cbase = base + ci * C
            pltpu.sync_copy(user_r.at[pl.ds(cbase, C)], idx_u)
            pltpu.sync_copy(item_r.at[pl.ds(cbase, C)], idx_i)
            cp0 = pltpu.async_copy(ui_r.at[idx_u], uiv, sem)
            cp1 = pltpu.async_copy(up_r.at[idx_u], upv, sem)
            cp2 = pltpu.async_copy(ii_r.at[idx_i], iiv, sem)
            cp3 = pltpu.async_copy(ip_r.at[idx_i], ipv, sem)
            cp0.wait()
            cp1.wait()
            cp2.wait()
            cp3.wait()
            for g in range(groups):
                res = jnp.zeros((lanes,), jnp.float32)
                for j in range(lanes):
                    p = g * lanes + j
                    acc = uiv[p, pl.ds(0, lanes)] * iiv[p, pl.ds(0, lanes)]
                    acc = acc + upv[p, pl.ds(0, lanes)] * ipv[p, pl.ds(0, lanes)]
                    for e in range(1, evecs):
                        s = pl.ds(e * lanes, lanes)
                        acc = acc + uiv[p, s] * iiv[p, s]
                        acc = acc + upv[p, s] * ipv[p, s]
                    for perm in xperms:
                        acc = acc + jnp.take_along_axis(acc, perm, axis=0)
                    res = jnp.where(lane == j, acc, res)
                outv[pl.ds(g * lanes, lanes)] = res
            pltpu.sync_copy(outv, out_r.at[pl.ds(cbase, C)])

    return dice


def kernel(user, item, users_int, users_pop, items_int, items_pop):
    B, L = user.shape
    E = users_int.shape[1]
    N = B * L
    info = plsc.get_sparse_core_info()
    n_workers = info.num_cores * info.num_subcores
    lanes = info.num_lanes

    dice = _build_dice_kernel(N, E, n_workers, lanes)
    out = dice(
        user.reshape(N).astype(jnp.int32),
        item.reshape(N).astype(jnp.int32),
        users_int, users_pop, items_int, items_pop,
    )
    return out.reshape(B, L)


# speedup vs baseline: 1.2246x; 1.2246x over previous
import functools

import jax
import jax.numpy as jnp
from jax import lax
from jax.experimental import pallas as pl
from jax.experimental.pallas import tpu as pltpu
from jax.experimental.pallas import tpu_sc as plsc


def _build_dice_kernel(N, E, n_workers, lanes):
    C = 128
    rows_total = N // C
    per_w = rows_total // n_workers
    groups = C // lanes
    evecs = E // lanes

    mesh = plsc.VectorSubcoreMesh(core_axis_name="c", subcore_axis_name="s")
    nc = mesh.num_cores

    @functools.partial(
        pl.kernel,
        out_type=jax.ShapeDtypeStruct((rows_total, C), jnp.float32),
        mesh=mesh,
        compiler_params=pltpu.CompilerParams(use_tc_tiling_on_sc=False),
        scratch_types=[
            pltpu.VMEM((per_w, C), jnp.int32),
            pltpu.VMEM((per_w, C), jnp.int32),
            [pltpu.VMEM((C, E), jnp.float32)] * 4,
            [pltpu.VMEM((C, E), jnp.float32)] * 4,
            pltpu.VMEM((per_w, C), jnp.float32),
            pltpu.SemaphoreType.DMA,
            pltpu.SemaphoreType.DMA,
        ],
    )
    def dice(user_r, item_r, ui_r, up_r, ii_r, ip_r, out_r,
             idx_u, idx_i, bufs_a, bufs_b, outv, sem_a, sem_b):
        wid = lax.axis_index("s") * nc + lax.axis_index("c")
        rbase = wid * per_w
        lane = lax.iota(jnp.int32, lanes)
        xperms = [lane ^ k for k in (8, 4, 2, 1)]
        tables = (ui_r, ii_r, up_r, ip_r)
        which_idx = (0, 1, 0, 1)

        pltpu.sync_copy(user_r.at[pl.ds(rbase, per_w)], idx_u)
        pltpu.sync_copy(item_r.at[pl.ds(rbase, per_w)], idx_i)

        def fire(c, bufs, sem):
            for tab, buf, w in zip(tables, bufs, which_idx):
                idx = idx_i if w else idx_u
                pltpu.async_copy(tab.at[idx.at[c]], buf, sem)

        def drain(c, bufs, sem):
            for tab, buf, w in zip(tables, bufs, which_idx):
                idx = idx_i if w else idx_u
                pltpu.make_async_copy(tab.at[idx.at[c]], buf, sem).wait()

        def compute(c, bufs):
            uiv, iiv, upv, ipv = bufs

            @pl.loop(0, groups)
            def gbody(g):
                res = jnp.zeros((lanes,), jnp.float32)
                for j in range(lanes):
                    p = g * lanes + j
                    acc = uiv[p, pl.ds(0, lanes)] * iiv[p, pl.ds(0, lanes)]
                    acc = acc + upv[p, pl.ds(0, lanes)] * ipv[p, pl.ds(0, lanes)]
                    for e in range(1, evecs):
                        s = pl.ds(e * lanes, lanes)
                        acc = acc + uiv[p, s] * iiv[p, s]
                        acc = acc + upv[p, s] * ipv[p, s]
                    for perm in xperms:
                        acc = acc + jnp.take_along_axis(acc, perm, axis=0)
                    res = jnp.where(lane == j, acc, res)
                outv[c, pl.ds(g * lanes, lanes)] = res

        fire(0, bufs_a, sem_a)

        @pl.loop(0, per_w // 2)
        def body(h):
            c0 = 2 * h
            c1 = c0 + 1
            fire(c1, bufs_b, sem_b)
            drain(c0, bufs_a, sem_a)
            compute(c0, bufs_a)
            cn = jnp.minimum(c1 + 1, per_w - 1)
            fire(cn, bufs_a, sem_a)
            drain(c1, bufs_b, sem_b)
            compute(c1, bufs_b)

        drain(per_w - 1, bufs_a, sem_a)
        pltpu.sync_copy(outv, out_r.at[pl.ds(rbase, per_w)])

    return dice


def kernel(user, item, users_int, users_pop, items_int, items_pop):
    B, L = user.shape
    E = users_int.shape[1]
    N = B * L
    info = plsc.get_sparse_core_info()
    n_workers = info.num_cores * info.num_subcores
    lanes = info.num_lanes
    C = 128
    rows_total = N // C

    dice = _build_dice_kernel(N, E, n_workers, lanes)
    out = dice(
        user.reshape(rows_total, C).astype(jnp.int32),
        item.reshape(rows_total, C).astype(jnp.int32),
        users_int, users_pop, items_int, items_pop,
    )
    return out.reshape(B, L)
